# baseline (speedup 1.0000x reference)
import functools

import jax
import jax.numpy as jnp
from jax import lax
from jax.experimental import pallas as pl
from jax.experimental.pallas import tpu as pltpu
from jax.experimental.pallas import tpu_sc as plsc

_N = 10000
_NP = 10240
_D = 128
_H = 32
_T = 50
_F = 3
_E = 320000
_NC = 2
_NS = 16
_NW = _NC * _NS
_RPW = 88
_EPW = _RPW * 128
_EPAD = _NW * _EPW
_CROWS = 4
_NCHUNK = _RPW // _CROWS
_STRIPE = _NP // _NS


def _gat_edges_sc(src3d, dst3d, a_src, a_dst, m128, h_ext):
    mesh = plsc.VectorSubcoreMesh(core_axis_name="c", subcore_axis_name="s",
                                  num_cores=_NC, num_subcores=_NS)

    @functools.partial(
        pl.kernel,
        out_type=[
            jax.ShapeDtypeStruct((_NC, _NP, _H), jnp.float32),
            jax.ShapeDtypeStruct((_NP,), jnp.float32),
            jax.ShapeDtypeStruct((_NP,), jnp.float32),
        ],
        mesh=mesh,
        compiler_params=pltpu.CompilerParams(needs_layout_passes=False,
                                             use_tc_tiling_on_sc=False),
        scratch_types=[
            pltpu.VMEM((_RPW, 1, 128), jnp.int32),
            pltpu.VMEM((_RPW, 1, 128), jnp.int32),
            pltpu.VMEM((_NP,), jnp.float32),
            pltpu.VMEM((_NP,), jnp.float32),
            pltpu.VMEM((16,), jnp.float32),
            pltpu.VMEM((_RPW, 1, 128), jnp.float32),
            pltpu.VMEM((2 * _CROWS * 128, _H), jnp.float32),
            pltpu.VMEM((_NP,), jnp.float32),
            pltpu.VMEM((_STRIPE,), jnp.float32),
            pltpu.VMEM_SHARED((_NP, _H), jnp.float32),
            pltpu.VMEM_SHARED((_NS, _NP), jnp.float32),
            pltpu.SemaphoreType.DMA,
            pltpu.SemaphoreType.DMA,
        ],
    )
    def k(src_hbm, dst_hbm, as_hbm, ad_hbm, m_hbm, h_hbm,
          acc_out, s0_out, s1_out,
          src_v, dst_v, as_v, ad_v, m_v, ex_v, rows_v, s_loc, sred,
          acc_sh, s16_sh, sem, sem2):
        cid = lax.axis_index("c")
        sid = lax.axis_index("s")
        wid = cid * _NS + sid
        base = wid * _RPW

        pltpu.sync_copy(src_hbm.at[pl.ds(base, _RPW)], src_v)
        pltpu.sync_copy(dst_hbm.at[pl.ds(base, _RPW)], dst_v)
        pltpu.sync_copy(as_hbm, as_v)
        pltpu.sync_copy(ad_hbm, ad_v)
        pltpu.sync_copy(m_hbm.at[pl.ds(0, 16)], m_v)

        def zs(i, c):
            s_loc[pl.ds(i * 16, 16)] = jnp.zeros((16,), jnp.float32)
            return c

        lax.fori_loop(0, _NP // 16, zs, 0)

        def zr(i, c):
            rows_v[i, pl.ds(0, 16)] = jnp.zeros((16,), jnp.float32)
            rows_v[i, pl.ds(16, 16)] = jnp.zeros((16,), jnp.float32)
            return c

        lax.fori_loop(0, _STRIPE, zr, 0)
        pltpu.sync_copy(rows_v.at[pl.ds(0, _STRIPE)],
                        acc_sh.at[pl.ds(sid * _STRIPE, _STRIPE)])
        plsc.subcore_barrier()

        m16 = m_v[...]

        def exrow(r, c):
            for kk in range(8):
                isrc = src_v[r, 0, pl.ds(kk * 16, 16)]
                idst = dst_v[r, 0, pl.ds(kk * 16, 16)]
                a = plsc.load_gather(as_v, [isrc]) + plsc.load_gather(ad_v, [idst])
                e = jnp.where(a > 0, a, a * jnp.float32(0.2))
                ex = jnp.exp(e - m16)
                ex_v[r, 0, pl.ds(kk * 16, 16)] = ex
                plsc.addupdate_scatter(s_loc, [idst], ex)
            return c

        lax.fori_loop(0, _RPW, exrow, 0)

        _CB = _CROWS * 128

        def _gather(cj, boff, rr):
            return pltpu.make_async_copy(
                h_hbm.at[src_v.at[cj * _CROWS + rr, 0]],
                rows_v.at[pl.ds(boff + rr * 128, 128)], sem)

        def _scatter(cj, boff, rr):
            return pltpu.make_async_copy(
                rows_v.at[pl.ds(boff + rr * 128, 128)],
                acc_sh.at[dst_v.at[cj * _CROWS + rr, 0]], sem2)

        def pipe(cj, c):
            boff = lax.rem(cj, 2) * _CB
            for rr in range(_CROWS):
                _gather(cj, boff, rr).wait()

            def scale(rr, cc):
                for kk in range(8):
                    exvec = ex_v[cj * _CROWS + rr, 0, pl.ds(kk * 16, 16)]
                    for jj in range(16):
                        e = exvec[jj]
                        j = boff + rr * 128 + kk * 16 + jj
                        rows_v[j, pl.ds(0, 16)] = rows_v[j, pl.ds(0, 16)] * e
                        rows_v[j, pl.ds(16, 16)] = rows_v[j, pl.ds(16, 16)] * e
                return cc

            lax.fori_loop(0, _CROWS, scale, 0)

            @pl.when(cj > 0)
            def _():
                for rr in range(_CROWS):
                    _scatter(cj - 1, _CB - boff, rr).wait()

            for rr in range(_CROWS):
                _scatter(cj, boff, rr).start(add=True)

            @pl.when(cj < _NCHUNK - 1)
            def _():
                for rr in range(_CROWS):
                    _gather(cj + 1, _CB - boff, rr).start()

            return c

        for rr in range(_CROWS):
            _gather(0, 0, rr).start()
        lax.fori_loop(0, _NCHUNK, pipe, 0)
        for rr in range(_CROWS):
            _scatter(_NCHUNK - 1, ((_NCHUNK - 1) % 2) * _CB, rr).wait()

        pltpu.sync_copy(s_loc, s16_sh.at[sid])
        plsc.subcore_barrier()
        for kt in range(_NS):
            pltpu.sync_copy(s16_sh.at[kt, pl.ds(sid * _STRIPE, _STRIPE)],
                            s_loc.at[pl.ds(kt * _STRIPE, _STRIPE)])

        def red(i, c):
            tot = s_loc[pl.ds(i * 16, 16)]
            for kt in range(1, _NS):
                tot = tot + s_loc[pl.ds(kt * _STRIPE + i * 16, 16)]
            sred[pl.ds(i * 16, 16)] = tot
            return c

        lax.fori_loop(0, _STRIPE // 16, red, 0)

        pltpu.sync_copy(acc_sh.at[pl.ds(sid * _STRIPE, _STRIPE)],
                        acc_out.at[cid, pl.ds(sid * _STRIPE, _STRIPE)])

        @pl.when(cid == 0)
        def _():
            pltpu.sync_copy(sred, s0_out.at[pl.ds(sid * _STRIPE, _STRIPE)])

        @pl.when(cid == 1)
        def _():
            pltpu.sync_copy(sred, s1_out.at[pl.ds(sid * _STRIPE, _STRIPE)])

    return k(src3d, dst3d, a_src, a_dst, m128, h_ext)


def _logit_rows(a_ref, h):
    af = lax.dot_general(a_ref[...], h, (((1,), (1,)), ((), ())),
                         preferred_element_type=jnp.float32)
    m = jnp.max(af[0:1, :]) + jnp.max(af[1:2, :])
    m = jnp.where(m > 0, m, m * 0.2)
    return af, m


def _prep_tc(x_pad, W1, A1):
    def body(x_ref, w_ref, a_ref, h_ref, as_ref, ad_ref, m_ref):
        h = jnp.dot(x_ref[...], w_ref[...], preferred_element_type=jnp.float32)
        h_ref[...] = h
        af, m = _logit_rows(a_ref, h)
        as_ref[...] = af[0]
        ad_ref[...] = af[1]
        m_ref[...] = jnp.full((128,), m, jnp.float32)

    return pl.pallas_call(
        body,
        out_shape=[
            jax.ShapeDtypeStruct((_NP, _H), jnp.float32),
            jax.ShapeDtypeStruct((_NP,), jnp.float32),
            jax.ShapeDtypeStruct((_NP,), jnp.float32),
            jax.ShapeDtypeStruct((128,), jnp.float32),
        ],
    )(x_pad, W1, A1)


def _mid_tc(acc, s0, s1, b1, W2, A2):
    def body(acc_ref, s0_ref, s1_ref, b1_ref, w2_ref, a2_ref,
             h2_ref, as_ref, ad_ref, m_ref):
        a = acc_ref[0] + acc_ref[1]
        s = s0_ref[...] + s1_ref[...]
        out1 = a / (s + 1e-16) + b1_ref[...]
        h1r = jnp.maximum(out1, 0.0)
        rows = lax.broadcasted_iota(jnp.int32, (_NP, _H), 0)
        h1r = jnp.where(rows < _N, h1r, 0.0)
        h2 = jnp.dot(h1r, w2_ref[...], preferred_element_type=jnp.float32)
        h2_ref[...] = h2
        af, m = _logit_rows(a2_ref, h2)
        as_ref[...] = af[0]
        ad_ref[...] = af[1]
        m_ref[...] = jnp.full((128,), m, jnp.float32)

    return pl.pallas_call(
        body,
        out_shape=[
            jax.ShapeDtypeStruct((_NP, _H), jnp.float32),
            jax.ShapeDtypeStruct((_NP,), jnp.float32),
            jax.ShapeDtypeStruct((_NP,), jnp.float32),
            jax.ShapeDtypeStruct((128,), jnp.float32),
        ],
    )(acc, s0, s1, b1, W2, A2)


_BN = 5120


def _lstm_tc(seqT, Wg, bgT, ftT, fcb):
    def body(seq_ref, wg_ref, bg_ref, ft_ref, fcb_ref, z_ref):
        wg = wg_ref[...]
        bgv = bg_ref[...]

        def step(t, carry):
            h, c = carry
            xt = seq_ref[t]
            zin = jnp.concatenate([xt, h], axis=0)
            g = jnp.dot(wg, zin, preferred_element_type=jnp.float32) + bgv
            i = jax.nn.sigmoid(g[0:_H, :])
            f = jax.nn.sigmoid(g[_H:2 * _H, :])
            gg = jnp.tanh(g[2 * _H:3 * _H, :])
            o = jax.nn.sigmoid(g[3 * _H:4 * _H, :])
            c = f * c + i * gg
            h = o * jnp.tanh(c)
            return (h, c)

        z0 = jnp.zeros((_H, _BN), jnp.float32)
        h, _ = lax.fori_loop(0, _T, step, (z0, z0))
        z = lax.dot_general(h, ft_ref[...], (((0,), (0,)), ((), ())),
                            preferred_element_type=jnp.float32)
        z_ref[...] = z + fcb_ref[...]

    return pl.pallas_call(
        body,
        grid=(_NP // _BN,),
        in_specs=[
            pl.BlockSpec((_T, 8, _BN), lambda i: (0, 0, i)),
            pl.BlockSpec((4 * _H, 8 + _H), lambda i: (0, 0)),
            pl.BlockSpec((4 * _H, 1), lambda i: (0, 0)),
            pl.BlockSpec((_H, 1), lambda i: (0, 0)),
            pl.BlockSpec((1, 1), lambda i: (0, 0)),
        ],
        out_specs=pl.BlockSpec((_BN, 1), lambda i: (i, 0)),
        out_shape=jax.ShapeDtypeStruct((_NP, 1), jnp.float32),
    )(seqT, Wg, bgT, ftT, fcb)


def _head_tc(acc, s0, s1, b2, fgT, zt):
    def body(acc_ref, s0_ref, s1_ref, b2_ref, fg_ref, zt_ref, o_ref):
        a = acc_ref[0] + acc_ref[1]
        s = s0_ref[...] + s1_ref[...]
        hg = a / (s + 1e-16) + b2_ref[...]
        z = jnp.dot(hg, fg_ref[...], preferred_element_type=jnp.float32)
        o_ref[...] = jax.nn.sigmoid(z + zt_ref[...])

    return pl.pallas_call(
        body,
        out_shape=jax.ShapeDtypeStruct((_NP, 1), jnp.float32),
    )(acc, s0, s1, b2, fgT, zt)


def kernel(x, edge_index, sequences, W1, as1, ad1, b1, W2, as2, ad2, b2,
           Wih, Whh, bih, bhh, fcW, fcb):
    f32 = jnp.float32
    x_pad = jnp.pad(x, ((0, _NP - _N), (0, 0)))
    A1 = jnp.zeros((8, _H), f32).at[0].set(as1).at[1].set(ad1)
    A2 = jnp.zeros((8, _H), f32).at[0].set(as2).at[1].set(ad2)

    loop = jnp.arange(_N, dtype=jnp.int32)
    npad_e = _EPAD - _E - _N
    pad_e = _N + (jnp.arange(npad_e, dtype=jnp.int32) % (_NP - _N))
    src = jnp.concatenate([edge_index[0], loop, pad_e]).reshape(
        _NW * _RPW, 1, 128)
    dst = jnp.concatenate([edge_index[1], loop, pad_e]).reshape(
        _NW * _RPW, 1, 128)

    seqT = jnp.pad(jnp.transpose(sequences, (1, 2, 0)),
                   ((0, 0), (0, 8 - _F), (0, _NP - _N)))
    Wg = jnp.concatenate([jnp.pad(Wih, ((0, 0), (0, 8 - _F))), Whh],
                         axis=1)
    bgT = (bih + bhh).reshape(4 * _H, 1)
    fgT = fcW[:, :_H].T
    ftT = fcW[:, _H:].T

    h1, as1f, ad1f, m1 = _prep_tc(x_pad, W1, A1)
    acc1, s10, s11 = _gat_edges_sc(src, dst, as1f, ad1f, m1, h1)
    h2, as2f, ad2f, m2 = _mid_tc(acc1, s10.reshape(_NP, 1),
                                 s11.reshape(_NP, 1),
                                 b1.reshape(1, _H), W2, A2)
    acc2, s20, s21 = _gat_edges_sc(src, dst, as2f, ad2f, m2, h2)

    zt = _lstm_tc(seqT, Wg, bgT, ftT, fcb.reshape(1, 1))

    out = _head_tc(acc2, s20.reshape(_NP, 1), s21.reshape(_NP, 1),
                   b2.reshape(1, _H), fgT, zt)
    return out[:_N]


# speedup vs baseline: 1.3781x; 1.3781x over previous
import functools

import jax
import jax.numpy as jnp
from jax import lax
from jax.experimental import pallas as pl
from jax.experimental.pallas import tpu as pltpu
from jax.experimental.pallas import tpu_sc as plsc

_N = 10000
_NP = 10240
_D = 128
_H = 32
_T = 50
_F = 3
_E = 320000
_NC = 2
_NS = 16
_NW = _NC * _NS
_RPW = 84
_EPW = _RPW * 128
_EPAD = _NW * _EPW
_CROWS = 6
_NCHUNK = _RPW // _CROWS
_STRIPE = _NP // _NS


def _gat_edges_sc(src3d, dst3d, a_src, a_dst, m128, h_ext):
    mesh = plsc.VectorSubcoreMesh(core_axis_name="c", subcore_axis_name="s",
                                  num_cores=_NC, num_subcores=_NS)

    @functools.partial(
        pl.kernel,
        out_type=[
            jax.ShapeDtypeStruct((_NC, _NP, _H), jnp.float32),
            jax.ShapeDtypeStruct((_NP,), jnp.float32),
            jax.ShapeDtypeStruct((_NP,), jnp.float32),
        ],
        mesh=mesh,
        compiler_params=pltpu.CompilerParams(needs_layout_passes=False,
                                             use_tc_tiling_on_sc=False),
        scratch_types=[
            pltpu.VMEM((_RPW, 1, 128), jnp.int32),
            pltpu.VMEM((_RPW, 1, 128), jnp.int32),
            pltpu.VMEM((_NP,), jnp.float32),
            pltpu.VMEM((_NP,), jnp.float32),
            pltpu.VMEM((16,), jnp.float32),
            pltpu.VMEM((_RPW, 1, 128), jnp.float32),
            pltpu.VMEM((_CROWS * 128, _H), jnp.float32),
            pltpu.VMEM((_NP,), jnp.float32),
            pltpu.VMEM((_STRIPE,), jnp.float32),
            pltpu.VMEM_SHARED((_NP, _H), jnp.float32),
            pltpu.VMEM_SHARED((_NS, _NP), jnp.float32),
            pltpu.SemaphoreType.DMA,
        ],
    )
    def k(src_hbm, dst_hbm, as_hbm, ad_hbm, m_hbm, h_hbm,
          acc_out, s0_out, s1_out,
          src_v, dst_v, as_v, ad_v, m_v, ex_v, rows_v, s_loc, sred,
          acc_sh, s16_sh, sem):
        cid = lax.axis_index("c")
        sid = lax.axis_index("s")
        wid = cid * _NS + sid
        base = wid * _RPW

        pltpu.sync_copy(src_hbm.at[pl.ds(base, _RPW)], src_v)
        pltpu.sync_copy(dst_hbm.at[pl.ds(base, _RPW)], dst_v)
        pltpu.sync_copy(as_hbm, as_v)
        pltpu.sync_copy(ad_hbm, ad_v)
        pltpu.sync_copy(m_hbm.at[pl.ds(0, 16)], m_v)

        def zs(i, c):
            s_loc[pl.ds(i * 16, 16)] = jnp.zeros((16,), jnp.float32)
            return c

        lax.fori_loop(0, _NP // 16, zs, 0)

        def zr(i, c):
            rows_v[i, pl.ds(0, 16)] = jnp.zeros((16,), jnp.float32)
            rows_v[i, pl.ds(16, 16)] = jnp.zeros((16,), jnp.float32)
            return c

        lax.fori_loop(0, _STRIPE, zr, 0)
        pltpu.sync_copy(rows_v.at[pl.ds(0, _STRIPE)],
                        acc_sh.at[pl.ds(sid * _STRIPE, _STRIPE)])
        plsc.subcore_barrier()

        m16 = m_v[...]

        def exrow(r, c):
            for kk in range(8):
                isrc = src_v[r, 0, pl.ds(kk * 16, 16)]
                idst = dst_v[r, 0, pl.ds(kk * 16, 16)]
                a = plsc.load_gather(as_v, [isrc]) + plsc.load_gather(ad_v, [idst])
                e = jnp.where(a > 0, a, a * jnp.float32(0.2))
                ex = jnp.exp(e - m16)
                ex_v[r, 0, pl.ds(kk * 16, 16)] = ex
                plsc.addupdate_scatter(s_loc, [idst], ex)
            return c

        lax.fori_loop(0, _RPW, exrow, 0)

        def chunk(cj, c):
            r0 = cj * _CROWS
            cps = [
                pltpu.async_copy(h_hbm.at[src_v.at[r0 + rr, 0]],
                                 rows_v.at[pl.ds(rr * 128, 128)], sem)
                for rr in range(_CROWS)
            ]
            for cp in cps:
                cp.wait()

            def scale(rr, cc):
                for kk in range(8):
                    exvec = ex_v[r0 + rr, 0, pl.ds(kk * 16, 16)]
                    for jj in range(16):
                        e = exvec[jj]
                        j = rr * 128 + kk * 16 + jj
                        rows_v[j, pl.ds(0, 16)] = rows_v[j, pl.ds(0, 16)] * e
                        rows_v[j, pl.ds(16, 16)] = rows_v[j, pl.ds(16, 16)] * e
                return cc

            lax.fori_loop(0, _CROWS, scale, 0)
            for rr in range(_CROWS):
                pltpu.sync_copy(rows_v.at[pl.ds(rr * 128, 128)],
                                acc_sh.at[dst_v.at[r0 + rr, 0]], add=True)
            return c

        lax.fori_loop(0, _NCHUNK, chunk, 0)

        pltpu.sync_copy(s_loc, s16_sh.at[sid])
        plsc.subcore_barrier()
        for kt in range(_NS):
            pltpu.sync_copy(s16_sh.at[kt, pl.ds(sid * _STRIPE, _STRIPE)],
                            s_loc.at[pl.ds(kt * _STRIPE, _STRIPE)])

        def red(i, c):
            tot = s_loc[pl.ds(i * 16, 16)]
            for kt in range(1, _NS):
                tot = tot + s_loc[pl.ds(kt * _STRIPE + i * 16, 16)]
            sred[pl.ds(i * 16, 16)] = tot
            return c

        lax.fori_loop(0, _STRIPE // 16, red, 0)

        pltpu.sync_copy(acc_sh.at[pl.ds(sid * _STRIPE, _STRIPE)],
                        acc_out.at[cid, pl.ds(sid * _STRIPE, _STRIPE)])

        @pl.when(cid == 0)
        def _():
            pltpu.sync_copy(sred, s0_out.at[pl.ds(sid * _STRIPE, _STRIPE)])

        @pl.when(cid == 1)
        def _():
            pltpu.sync_copy(sred, s1_out.at[pl.ds(sid * _STRIPE, _STRIPE)])

    return k(src3d, dst3d, a_src, a_dst, m128, h_ext)


def _logit_rows(a_ref, h):
    af = lax.dot_general(a_ref[...], h, (((1,), (1,)), ((), ())),
                         preferred_element_type=jnp.float32)
    m = jnp.max(af[0:1, :]) + jnp.max(af[1:2, :])
    m = jnp.where(m > 0, m, m * 0.2)
    return af, m


def _prep_tc(x_pad, W1, A1):
    def body(x_ref, w_ref, a_ref, h_ref, as_ref, ad_ref, m_ref):
        h = jnp.dot(x_ref[...], w_ref[...], preferred_element_type=jnp.float32)
        h_ref[...] = h
        af, m = _logit_rows(a_ref, h)
        as_ref[...] = af[0]
        ad_ref[...] = af[1]
        m_ref[...] = jnp.full((128,), m, jnp.float32)

    return pl.pallas_call(
        body,
        out_shape=[
            jax.ShapeDtypeStruct((_NP, _H), jnp.float32),
            jax.ShapeDtypeStruct((_NP,), jnp.float32),
            jax.ShapeDtypeStruct((_NP,), jnp.float32),
            jax.ShapeDtypeStruct((128,), jnp.float32),
        ],
    )(x_pad, W1, A1)


def _mid_tc(acc, s0, s1, b1, W2, A2):
    def body(acc_ref, s0_ref, s1_ref, b1_ref, w2_ref, a2_ref,
             h2_ref, as_ref, ad_ref, m_ref):
        a = acc_ref[0] + acc_ref[1]
        s = s0_ref[...] + s1_ref[...]
        out1 = a / (s + 1e-16) + b1_ref[...]
        h1r = jnp.maximum(out1, 0.0)
        rows = lax.broadcasted_iota(jnp.int32, (_NP, _H), 0)
        h1r = jnp.where(rows < _N, h1r, 0.0)
        h2 = jnp.dot(h1r, w2_ref[...], preferred_element_type=jnp.float32)
        h2_ref[...] = h2
        af, m = _logit_rows(a2_ref, h2)
        as_ref[...] = af[0]
        ad_ref[...] = af[1]
        m_ref[...] = jnp.full((128,), m, jnp.float32)

    return pl.pallas_call(
        body,
        out_shape=[
            jax.ShapeDtypeStruct((_NP, _H), jnp.float32),
            jax.ShapeDtypeStruct((_NP,), jnp.float32),
            jax.ShapeDtypeStruct((_NP,), jnp.float32),
            jax.ShapeDtypeStruct((128,), jnp.float32),
        ],
    )(acc, s0, s1, b1, W2, A2)


_BN = 5120


def _lstm_tc(seqT, Wg, bgT, ftT, fcb):
    def body(seq_ref, wg_ref, bg_ref, ft_ref, fcb_ref, z_ref):
        wg = wg_ref[...]
        bgv = bg_ref[...]

        def step(t, carry):
            h, c = carry
            xt = seq_ref[t]
            zin = jnp.concatenate([xt, h], axis=0)
            g = jnp.dot(wg, zin, preferred_element_type=jnp.float32) + bgv
            i = jax.nn.sigmoid(g[0:_H, :])
            f = jax.nn.sigmoid(g[_H:2 * _H, :])
            gg = jnp.tanh(g[2 * _H:3 * _H, :])
            o = jax.nn.sigmoid(g[3 * _H:4 * _H, :])
            c = f * c + i * gg
            h = o * jnp.tanh(c)
            return (h, c)

        z0 = jnp.zeros((_H, _BN), jnp.float32)
        h, _ = lax.fori_loop(0, _T, step, (z0, z0))
        z = lax.dot_general(h, ft_ref[...], (((0,), (0,)), ((), ())),
                            preferred_element_type=jnp.float32)
        z_ref[...] = z + fcb_ref[...]

    return pl.pallas_call(
        body,
        grid=(_NP // _BN,),
        in_specs=[
            pl.BlockSpec((_T, 8, _BN), lambda i: (0, 0, i)),
            pl.BlockSpec((4 * _H, 8 + _H), lambda i: (0, 0)),
            pl.BlockSpec((4 * _H, 1), lambda i: (0, 0)),
            pl.BlockSpec((_H, 1), lambda i: (0, 0)),
            pl.BlockSpec((1, 1), lambda i: (0, 0)),
        ],
        out_specs=pl.BlockSpec((_BN, 1), lambda i: (i, 0)),
        out_shape=jax.ShapeDtypeStruct((_NP, 1), jnp.float32),
    )(seqT, Wg, bgT, ftT, fcb)


def _head_tc(acc, s0, s1, b2, fgT, zt):
    def body(acc_ref, s0_ref, s1_ref, b2_ref, fg_ref, zt_ref, o_ref):
        a = acc_ref[0] + acc_ref[1]
        s = s0_ref[...] + s1_ref[...]
        hg = a / (s + 1e-16) + b2_ref[...]
        z = jnp.dot(hg, fg_ref[...], preferred_element_type=jnp.float32)
        o_ref[...] = jax.nn.sigmoid(z + zt_ref[...])

    return pl.pallas_call(
        body,
        out_shape=jax.ShapeDtypeStruct((_NP, 1), jnp.float32),
    )(acc, s0, s1, b2, fgT, zt)


def kernel(x, edge_index, sequences, W1, as1, ad1, b1, W2, as2, ad2, b2,
           Wih, Whh, bih, bhh, fcW, fcb):
    f32 = jnp.float32
    x_pad = jnp.pad(x, ((0, _NP - _N), (0, 0)))
    A1 = jnp.zeros((8, _H), f32).at[0].set(as1).at[1].set(ad1)
    A2 = jnp.zeros((8, _H), f32).at[0].set(as2).at[1].set(ad2)

    loop = jnp.arange(_N, dtype=jnp.int32)
    npad_e = _EPAD - _E - _N
    pad_e = _N + (jnp.arange(npad_e, dtype=jnp.int32) % (_NP - _N))
    src = jnp.concatenate([edge_index[0], loop, pad_e]).reshape(
        _NW * _RPW, 1, 128)
    dst = jnp.concatenate([edge_index[1], loop, pad_e]).reshape(
        _NW * _RPW, 1, 128)

    seqT = jnp.pad(jnp.transpose(sequences, (1, 2, 0)),
                   ((0, 0), (0, 8 - _F), (0, _NP - _N)))
    Wg = jnp.concatenate([jnp.pad(Wih, ((0, 0), (0, 8 - _F))), Whh],
                         axis=1)
    bgT = (bih + bhh).reshape(4 * _H, 1)
    fgT = fcW[:, :_H].T
    ftT = fcW[:, _H:].T

    h1, as1f, ad1f, m1 = _prep_tc(x_pad, W1, A1)
    acc1, s10, s11 = _gat_edges_sc(src, dst, as1f, ad1f, m1, h1)
    h2, as2f, ad2f, m2 = _mid_tc(acc1, s10.reshape(_NP, 1),
                                 s11.reshape(_NP, 1),
                                 b1.reshape(1, _H), W2, A2)
    acc2, s20, s21 = _gat_edges_sc(src, dst, as2f, ad2f, m2, h2)

    zt = _lstm_tc(seqT, Wg, bgT, ftT, fcb.reshape(1, 1))

    out = _head_tc(acc2, s20.reshape(_NP, 1), s21.reshape(_NP, 1),
                   b2.reshape(1, _H), fgT, zt)
    return out[:_N]
